# Initial kernel scaffold; baseline (speedup 1.0000x reference)
import jax
import jax.numpy as jnp
from jax.experimental import pallas as pl


def kernel(x):
    raise NotImplementedError("write your pallas kernel here")


# speedup vs baseline: 6.8878x; 6.8878x over previous
import functools

import jax
import jax.numpy as jnp
from jax import lax
from jax.experimental import pallas as pl
from jax.experimental.pallas import tpu as pltpu

_TEMP = 2.0
_PROB_MSK = (0, 2, 3)
_ENT_TEMP = 0.01

_B, _T, _F = 8, 1024, 512
_TBLK = 128
_NSTEP = _T // _TBLK
_NTOK = _B * _T


def _vq_body(x_ref, sub_ref, k_ref, ent_ref, cp_ref, pp_ref, dl_ref,
             avg_acc, cnt_acc):
    i = pl.program_id(0)

    @pl.when(i == 0)
    def _init():
        avg_acc[...] = jnp.zeros_like(avg_acc)
        cnt_acc[...] = jnp.zeros_like(cnt_acc)

    xb = x_ref[...]
    col = lax.broadcasted_iota(jnp.int32, xb.shape, 2)
    is_msk = (col == 0) | (col == 2) | (col == 3)
    xm = jnp.where(is_msk, -1e30, xb)

    m = jnp.max(xm, axis=2)
    d = jnp.maximum(xm - m[:, :, None], -80.0)
    e = jnp.exp(d)
    s0 = jnp.sum(e, axis=2)
    s1 = jnp.sum(e * d, axis=2)
    ent = jnp.log(s0) - s1 / s0
    ent_ref[i, :] = jnp.mean(ent, axis=0)

    kidx = jnp.min(jnp.where(xm == m[:, :, None], col, _F), axis=2)
    h = (col == kidx[:, :, None]).astype(jnp.float32)
    sub_ref[...] = h
    k_ref[:, pl.ds(i * _TBLK, _TBLK)] = kidx

    inv_s0 = 1.0 / s0
    avg_acc[0, :] += jnp.sum(e * inv_s0[:, :, None], axis=(0, 1))
    cnt_acc[0, :] += jnp.sum(h, axis=(0, 1))

    @pl.when(i == _NSTEP - 1)
    def _fini():
        avg = avg_acc[0, :] * (1.0 / _NTOK)
        hardp = cnt_acc[0, :] * (1.0 / _NTOK)
        cp = jnp.exp(-jnp.sum(hardp * jnp.log(hardp + 1e-7)))
        ppl = jnp.exp(-jnp.sum(avg * jnp.log(avg + 1e-7)))
        cp_ref[0, 0] = cp
        pp_ref[0, 0] = ppl
        dl_ref[0, 0] = ((_F - ppl) / _F) / _ENT_TEMP


@functools.partial(jax.jit, static_argnames=("interpret",))
def _vq_call(x, interpret=False):
    out_shapes = (
        jax.ShapeDtypeStruct((_B, _T, _F), jnp.float32),
        jax.ShapeDtypeStruct((_B, _T), jnp.int32),
        jax.ShapeDtypeStruct((_NSTEP, _TBLK), jnp.float32),
        jax.ShapeDtypeStruct((1, 1), jnp.float32),
        jax.ShapeDtypeStruct((1, 1), jnp.float32),
        jax.ShapeDtypeStruct((1, 1), jnp.float32),
    )
    grid = (_NSTEP,)
    in_specs = [pl.BlockSpec((_B, _TBLK, _F), lambda i: (0, i, 0))]
    out_specs = (
        pl.BlockSpec((_B, _TBLK, _F), lambda i: (0, i, 0)),
        pl.BlockSpec((_B, _T), lambda i: (0, 0)),
        pl.BlockSpec((_NSTEP, _TBLK), lambda i: (0, 0)),
        pl.BlockSpec(memory_space=pltpu.SMEM),
        pl.BlockSpec(memory_space=pltpu.SMEM),
        pl.BlockSpec(memory_space=pltpu.SMEM),
    )
    scratch = [
        pltpu.VMEM((1, _F), jnp.float32),
        pltpu.VMEM((1, _F), jnp.float32),
    ]
    return pl.pallas_call(
        _vq_body,
        grid=grid,
        in_specs=in_specs,
        out_specs=out_specs,
        out_shape=out_shapes,
        scratch_shapes=scratch,
        interpret=interpret,
    )(x)


def kernel(x):
    sub, k, ent, cp, ppl, dl = _vq_call(x)
    subword_prob = sub
    targets = k.reshape(_B, _T, 1)
    ent_per_t = ent.reshape(_T)
    code_perplexity = cp.reshape(())
    prob_perplexity = ppl.reshape(())
    diversity_loss = dl.reshape(())
    return (subword_prob, targets, code_perplexity, prob_perplexity,
            ent_per_t, diversity_loss)
